# baseline (speedup 1.0000x reference)
import functools

import jax
import jax.numpy as jnp
from jax import lax
from jax.experimental import pallas as pl
from jax.experimental.pallas import tpu as pltpu

_H = 12
_NEG = -1e30


def _rollout_body(a_ref, sel_ref, res_ref, *, N, L, k_keep, n_sel):
    t = pl.program_id(1)
    acc = a_ref[0, 0, 0]
    for h in range(1, _H):
        acc = acc + a_ref[0, 0, h]
    fused = acc / jnp.float32(_H)

    cols = lax.broadcasted_iota(jnp.int32, (N, N), 1)
    rows = lax.broadcasted_iota(jnp.int32, (N, N), 0)

    selected = jnp.zeros((N, N), jnp.float32)
    for _ in range(k_keep):
        cur = jnp.where(selected > 0.0, _NEG, fused)
        m = jnp.max(cur, axis=1, keepdims=True)
        selected = jnp.where(cur == m, 1.0, selected)
    mask = jnp.where(cols == 0, 1.0, selected)

    eye = jnp.where(rows == cols, 1.0, 0.0)
    md = fused * mask + eye
    M = md / jnp.sum(md, axis=1, keepdims=True)

    @pl.when(t == 0)
    def _():
        res_ref[...] = M

    @pl.when(t > 0)
    def _():
        res_ref[...] = jnp.dot(M, res_ref[...],
                               preferred_element_type=jnp.float32)

    @pl.when(t == L - 1)
    def _():
        v = res_ref[0:1, :]
        cid = lax.broadcasted_iota(jnp.int32, (1, N), 1)
        work = jnp.where(cid == 0, _NEG, v)
        for k in range(n_sel):
            m = jnp.max(work)
            idx = jnp.min(jnp.where(work == m, cid, jnp.int32(2 * N)))
            sel_ref[0, 0, k] = idx
            work = jnp.where(cid == idx, _NEG, work)


def _attn_body(sel_ref, x_ref, wq_ref, bq_ref, wk_ref, bk_ref, wv_ref, bv_ref,
               wp_ref, bp_ref, out_ref, *, N, C, n_sel):
    dh = C // _H
    scale = dh ** -0.5
    xb = x_ref[0]

    qrows = [x_ref[0, pl.ds(sel_ref[0, 0, k], 1), :] for k in range(n_sel)]
    qin = jnp.concatenate(qrows, axis=0)
    ql = jnp.dot(qin, wq_ref[...], preferred_element_type=jnp.float32) + bq_ref[0]

    outs = []
    for h in range(_H):
        sl = slice(h * dh, (h + 1) * dh)
        kg_h = jnp.dot(xb, wk_ref[:, sl], preferred_element_type=jnp.float32) + bk_ref[0, sl]
        vg_h = jnp.dot(xb, wv_ref[:, sl], preferred_element_type=jnp.float32) + bv_ref[0, sl]
        s = lax.dot_general(ql[:, sl], kg_h, (((1,), (1,)), ((), ())),
                            preferred_element_type=jnp.float32) * scale
        s = s - jnp.max(s, axis=1, keepdims=True)
        e = jnp.exp(s)
        w = e / jnp.sum(e, axis=1, keepdims=True)
        outs.append(jnp.dot(w, vg_h, preferred_element_type=jnp.float32))
    o = jnp.concatenate(outs, axis=1)
    outp = jnp.dot(o, wp_ref[...], preferred_element_type=jnp.float32) + bp_ref[0]

    out_ref[0] = jnp.zeros((N, C), jnp.float32)
    out_ref[0, 0:1, :] = x_ref[0, 0:1, :]
    for k in range(n_sel):
        out_ref[0, pl.ds(sel_ref[0, 0, k], 1), :] = outp[k:k + 1, :]


def kernel(x, attention_history, Wq, bq, Wk, bk, Wv, bv, Wp, bp):
    B, N, C = x.shape
    L = attention_history.shape[0]
    H = attention_history.shape[2]
    k_keep = N - int(N * 0.9)
    n_sel = max(1, int((N - 1) * 0.1))

    sel = pl.pallas_call(
        functools.partial(_rollout_body, N=N, L=L, k_keep=k_keep, n_sel=n_sel),
        grid=(B, L),
        in_specs=[
            pl.BlockSpec((1, 1, H, N, N), lambda b, t: (t, b, 0, 0, 0)),
        ],
        out_specs=pl.BlockSpec((1, 1, n_sel), lambda b, t: (b, 0, 0),
                               memory_space=pltpu.SMEM),
        out_shape=jax.ShapeDtypeStruct((B, 1, n_sel), jnp.int32),
        scratch_shapes=[pltpu.VMEM((N, N), jnp.float32)],
        compiler_params=pltpu.CompilerParams(
            dimension_semantics=("parallel", "arbitrary")),
    )(attention_history)

    out = pl.pallas_call(
        functools.partial(_attn_body, N=N, C=C, n_sel=n_sel),
        grid=(B,),
        in_specs=[
            pl.BlockSpec((1, 1, n_sel), lambda b: (b, 0, 0),
                         memory_space=pltpu.SMEM),
            pl.BlockSpec((1, N, C), lambda b: (b, 0, 0)),
            pl.BlockSpec((C, C), lambda b: (0, 0)),
            pl.BlockSpec((1, C), lambda b: (0, 0)),
            pl.BlockSpec((C, C), lambda b: (0, 0)),
            pl.BlockSpec((1, C), lambda b: (0, 0)),
            pl.BlockSpec((C, C), lambda b: (0, 0)),
            pl.BlockSpec((1, C), lambda b: (0, 0)),
            pl.BlockSpec((C, C), lambda b: (0, 0)),
            pl.BlockSpec((1, C), lambda b: (0, 0)),
        ],
        out_specs=pl.BlockSpec((1, N, C), lambda b: (b, 0, 0)),
        out_shape=jax.ShapeDtypeStruct((B, N, C), jnp.float32),
        compiler_params=pltpu.CompilerParams(
            dimension_semantics=("parallel",)),
    )(sel, x, Wq, bq.reshape(1, C), Wk, bk.reshape(1, C),
      Wv, bv.reshape(1, C), Wp, bp.reshape(1, C))

    return out


# speedup vs baseline: 1.0047x; 1.0047x over previous
import functools

import jax
import jax.numpy as jnp
from jax import lax
from jax.experimental import pallas as pl
from jax.experimental.pallas import tpu as pltpu

_H = 12
_NEG = -1e30


def _rollout_body(a_ref, row3_ref, sel_ref, res_ref, *, N, L, k_keep, n_sel):
    t = pl.program_id(1)
    acc = a_ref[0, 0, 0]
    for h in range(1, _H):
        acc = acc + a_ref[0, 0, h]
    fused = acc / jnp.float32(_H)

    cols = lax.broadcasted_iota(jnp.int32, (N, N), 1)
    rows = lax.broadcasted_iota(jnp.int32, (N, N), 0)

    work = fused
    for _ in range(k_keep):
        m = jnp.max(work, axis=1, keepdims=True)
        work = jnp.where(work == m, _NEG, work)
    mask = jnp.where(cols == 0, 1.0, jnp.where(work == _NEG, 1.0, 0.0))

    eye = jnp.where(rows == cols, 1.0, 0.0)
    md = fused * mask + eye
    M = md / jnp.sum(md, axis=1, keepdims=True)

    @pl.when(t == 0)
    def _():
        res_ref[...] = M

    @pl.when(t > 0)
    def _():
        res_ref[...] = jnp.dot(M, res_ref[...],
                               preferred_element_type=jnp.float32)

    @pl.when(t == L - 2)
    def _():
        racc = row3_ref[0, 0, 0:1, :]
        for h in range(1, _H):
            racc = racc + row3_ref[0, 0, h:h + 1, :]
        frow = racc / jnp.float32(_H)

        cid = lax.broadcasted_iota(jnp.int32, (1, N), 1)
        rwork = frow
        for _ in range(k_keep):
            m = jnp.max(rwork, axis=1, keepdims=True)
            rwork = jnp.where(rwork == m, _NEG, rwork)
        rmask = jnp.where(cid == 0, 1.0, jnp.where(rwork == _NEG, 1.0, 0.0))
        e0 = jnp.where(cid == 0, 1.0, 0.0)
        mdr = frow * rmask + e0
        v = mdr / jnp.sum(mdr)

        scores = jnp.dot(v, res_ref[...], preferred_element_type=jnp.float32)
        swork = jnp.where(cid == 0, _NEG, scores)
        for k in range(n_sel):
            m = jnp.max(swork)
            idx = jnp.min(jnp.where(swork == m, cid, jnp.int32(2 * N)))
            sel_ref[0, 0, k] = idx
            swork = jnp.where(cid == idx, _NEG, swork)


def _attn_body(sel_ref, x_ref, wq_ref, bq_ref, wk_ref, bk_ref, wv_ref, bv_ref,
               wp_ref, bp_ref, out_ref, *, N, C, n_sel):
    dh = C // _H
    scale = dh ** -0.5
    xb = x_ref[0]

    qrows = [x_ref[0, pl.ds(sel_ref[0, 0, k], 1), :] for k in range(n_sel)]
    qin = jnp.concatenate(qrows, axis=0)
    ql = jnp.dot(qin, wq_ref[...], preferred_element_type=jnp.float32) + bq_ref[0]

    outs = []
    for h in range(_H):
        sl = slice(h * dh, (h + 1) * dh)
        kg_h = jnp.dot(xb, wk_ref[:, sl], preferred_element_type=jnp.float32) + bk_ref[0, sl]
        vg_h = jnp.dot(xb, wv_ref[:, sl], preferred_element_type=jnp.float32) + bv_ref[0, sl]
        s = lax.dot_general(ql[:, sl], kg_h, (((1,), (1,)), ((), ())),
                            preferred_element_type=jnp.float32) * scale
        s = s - jnp.max(s, axis=1, keepdims=True)
        e = jnp.exp(s)
        w = e / jnp.sum(e, axis=1, keepdims=True)
        outs.append(jnp.dot(w, vg_h, preferred_element_type=jnp.float32))
    o = jnp.concatenate(outs, axis=1)
    outp = jnp.dot(o, wp_ref[...], preferred_element_type=jnp.float32) + bp_ref[0]

    out_ref[0] = jnp.zeros((N, C), jnp.float32)
    out_ref[0, 0:1, :] = x_ref[0, 0:1, :]
    for k in range(n_sel):
        out_ref[0, pl.ds(sel_ref[0, 0, k], 1), :] = outp[k:k + 1, :]


def kernel(x, attention_history, Wq, bq, Wk, bk, Wv, bv, Wp, bp):
    B, N, C = x.shape
    L = attention_history.shape[0]
    H = attention_history.shape[2]
    k_keep = N - int(N * 0.9)
    n_sel = max(1, int((N - 1) * 0.1))

    row3 = attention_history[L - 1, :, :, 0, :].reshape(B, 1, H, N)

    sel = pl.pallas_call(
        functools.partial(_rollout_body, N=N, L=L, k_keep=k_keep, n_sel=n_sel),
        grid=(B, L - 1),
        in_specs=[
            pl.BlockSpec((1, 1, H, N, N), lambda b, t: (t, b, 0, 0, 0)),
            pl.BlockSpec((1, 1, H, N), lambda b, t: (b, 0, 0, 0)),
        ],
        out_specs=pl.BlockSpec((1, 1, n_sel), lambda b, t: (b, 0, 0),
                               memory_space=pltpu.SMEM),
        out_shape=jax.ShapeDtypeStruct((B, 1, n_sel), jnp.int32),
        scratch_shapes=[pltpu.VMEM((N, N), jnp.float32)],
        compiler_params=pltpu.CompilerParams(
            dimension_semantics=("parallel", "arbitrary")),
    )(attention_history, row3)

    out = pl.pallas_call(
        functools.partial(_attn_body, N=N, C=C, n_sel=n_sel),
        grid=(B,),
        in_specs=[
            pl.BlockSpec((1, 1, n_sel), lambda b: (b, 0, 0),
                         memory_space=pltpu.SMEM),
            pl.BlockSpec((1, N, C), lambda b: (b, 0, 0)),
            pl.BlockSpec((C, C), lambda b: (0, 0)),
            pl.BlockSpec((1, C), lambda b: (0, 0)),
            pl.BlockSpec((C, C), lambda b: (0, 0)),
            pl.BlockSpec((1, C), lambda b: (0, 0)),
            pl.BlockSpec((C, C), lambda b: (0, 0)),
            pl.BlockSpec((1, C), lambda b: (0, 0)),
            pl.BlockSpec((C, C), lambda b: (0, 0)),
            pl.BlockSpec((1, C), lambda b: (0, 0)),
        ],
        out_specs=pl.BlockSpec((1, N, C), lambda b: (b, 0, 0)),
        out_shape=jax.ShapeDtypeStruct((B, N, C), jnp.float32),
        compiler_params=pltpu.CompilerParams(
            dimension_semantics=("parallel",)),
    )(sel, x, Wq, bq.reshape(1, C), Wk, bk.reshape(1, C),
      Wv, bv.reshape(1, C), Wp, bp.reshape(1, C))

    return out


# speedup vs baseline: 1.1937x; 1.1881x over previous
import functools

import jax
import jax.numpy as jnp
from jax import lax
from jax.experimental import pallas as pl
from jax.experimental.pallas import tpu as pltpu

_H = 12
_NEG = -1e30


_SEG = 200


def _rollout_body(a_ref, row3_ref, sel_ref, w_ref, *, N, L, k_keep, n_sel):
    nl = L - 1
    fused = []
    for l in range(nl):
        acc = a_ref[l, 0, 0]
        for h in range(1, _H):
            acc = acc + a_ref[l, 0, h]
        fused.append(acc / jnp.float32(_H))
    racc = row3_ref[0, 0, 0:1, :]
    for h in range(1, _H):
        racc = racc + row3_ref[0, 0, h:h + 1, :]
    frow = racc / jnp.float32(_H)

    for l in range(nl):
        w_ref[l * _SEG:l * _SEG + N, :] = fused[l]
        w_ref[l * _SEG + N:(l + 1) * _SEG, :] = jnp.zeros((_SEG - N, N),
                                                          jnp.float32)
    w_ref[nl * _SEG:nl * _SEG + 1, :] = frow
    w_ref[nl * _SEG + 1:nl * _SEG + 8, :] = jnp.zeros((7, N), jnp.float32)

    work = w_ref[...]
    for _ in range(k_keep):
        m = jnp.max(work, axis=1, keepdims=True)
        work = jnp.where(work == m, _NEG, work)
    cols = lax.broadcasted_iota(jnp.int32, work.shape, 1)
    maskc = jnp.where(cols == 0, 1.0, jnp.where(work == _NEG, 1.0, 0.0))

    rows = lax.broadcasted_iota(jnp.int32, (N, N), 0)
    colsN = lax.broadcasted_iota(jnp.int32, (N, N), 1)
    eye = jnp.where(rows == colsN, 1.0, 0.0)
    res = None
    for l in range(nl):
        md = fused[l] * maskc[l * _SEG:l * _SEG + N, :] + eye
        M = md / jnp.sum(md, axis=1, keepdims=True)
        if res is None:
            res = M
        else:
            res = jnp.dot(M, res, preferred_element_type=jnp.float32)

    cid = lax.broadcasted_iota(jnp.int32, (1, N), 1)
    e0 = jnp.where(cid == 0, 1.0, 0.0)
    mdr = frow * maskc[nl * _SEG:nl * _SEG + 1, :] + e0
    v = mdr / jnp.sum(mdr)

    scores = jnp.dot(v, res, preferred_element_type=jnp.float32)
    swork = jnp.where(cid == 0, _NEG, scores)
    for k in range(n_sel):
        m = jnp.max(swork)
        idx = jnp.min(jnp.where(swork == m, cid, jnp.int32(2 * N)))
        sel_ref[0, 0, k] = idx
        swork = jnp.where(cid == idx, _NEG, swork)


def _attn_body(sel_ref, x_ref, wq_ref, bq_ref, wk_ref, bk_ref, wv_ref, bv_ref,
               wp_ref, bp_ref, out_ref, *, N, C, n_sel):
    dh = C // _H
    scale = dh ** -0.5
    xb = x_ref[0]

    qrows = [x_ref[0, pl.ds(sel_ref[0, 0, k], 1), :] for k in range(n_sel)]
    qin = jnp.concatenate(qrows, axis=0)
    ql = jnp.dot(qin, wq_ref[...], preferred_element_type=jnp.float32) + bq_ref[0]

    outs = []
    for h in range(_H):
        sl = slice(h * dh, (h + 1) * dh)
        kg_h = jnp.dot(xb, wk_ref[:, sl], preferred_element_type=jnp.float32) + bk_ref[0, sl]
        vg_h = jnp.dot(xb, wv_ref[:, sl], preferred_element_type=jnp.float32) + bv_ref[0, sl]
        s = lax.dot_general(ql[:, sl], kg_h, (((1,), (1,)), ((), ())),
                            preferred_element_type=jnp.float32) * scale
        s = s - jnp.max(s, axis=1, keepdims=True)
        e = jnp.exp(s)
        w = e / jnp.sum(e, axis=1, keepdims=True)
        outs.append(jnp.dot(w, vg_h, preferred_element_type=jnp.float32))
    o = jnp.concatenate(outs, axis=1)
    outp = jnp.dot(o, wp_ref[...], preferred_element_type=jnp.float32) + bp_ref[0]

    out_ref[0] = jnp.zeros((N, C), jnp.float32)
    out_ref[0, 0:1, :] = x_ref[0, 0:1, :]
    for k in range(n_sel):
        out_ref[0, pl.ds(sel_ref[0, 0, k], 1), :] = outp[k:k + 1, :]


def kernel(x, attention_history, Wq, bq, Wk, bk, Wv, bv, Wp, bp):
    B, N, C = x.shape
    L = attention_history.shape[0]
    H = attention_history.shape[2]
    k_keep = N - int(N * 0.9)
    n_sel = max(1, int((N - 1) * 0.1))

    row3 = attention_history[L - 1, :, :, 0, :].reshape(B, 1, H, N)

    sel = pl.pallas_call(
        functools.partial(_rollout_body, N=N, L=L, k_keep=k_keep, n_sel=n_sel),
        grid=(B,),
        in_specs=[
            pl.BlockSpec((L - 1, 1, H, N, N), lambda b: (0, b, 0, 0, 0)),
            pl.BlockSpec((1, 1, H, N), lambda b: (b, 0, 0, 0)),
        ],
        out_specs=pl.BlockSpec((1, 1, n_sel), lambda b: (b, 0, 0),
                               memory_space=pltpu.SMEM),
        out_shape=jax.ShapeDtypeStruct((B, 1, n_sel), jnp.int32),
        scratch_shapes=[pltpu.VMEM(((L - 1) * _SEG + 8, N), jnp.float32)],
        compiler_params=pltpu.CompilerParams(
            dimension_semantics=("parallel",)),
    )(attention_history, row3)

    out = pl.pallas_call(
        functools.partial(_attn_body, N=N, C=C, n_sel=n_sel),
        grid=(B,),
        in_specs=[
            pl.BlockSpec((1, 1, n_sel), lambda b: (b, 0, 0),
                         memory_space=pltpu.SMEM),
            pl.BlockSpec((1, N, C), lambda b: (b, 0, 0)),
            pl.BlockSpec((C, C), lambda b: (0, 0)),
            pl.BlockSpec((1, C), lambda b: (0, 0)),
            pl.BlockSpec((C, C), lambda b: (0, 0)),
            pl.BlockSpec((1, C), lambda b: (0, 0)),
            pl.BlockSpec((C, C), lambda b: (0, 0)),
            pl.BlockSpec((1, C), lambda b: (0, 0)),
            pl.BlockSpec((C, C), lambda b: (0, 0)),
            pl.BlockSpec((1, C), lambda b: (0, 0)),
        ],
        out_specs=pl.BlockSpec((1, N, C), lambda b: (b, 0, 0)),
        out_shape=jax.ShapeDtypeStruct((B, N, C), jnp.float32),
        compiler_params=pltpu.CompilerParams(
            dimension_semantics=("parallel",)),
    )(sel, x, Wq, bq.reshape(1, C), Wk, bk.reshape(1, C),
      Wv, bv.reshape(1, C), Wp, bp.reshape(1, C))

    return out
